# Initial kernel scaffold; baseline (speedup 1.0000x reference)
import jax
import jax.numpy as jnp
from jax.experimental import pallas as pl


def kernel(x, geo_features, neighbor_indices):
    raise NotImplementedError("write your pallas kernel here")


# speedup vs baseline: 1.5024x; 1.5024x over previous
import functools

import jax
import jax.numpy as jnp
from jax import lax
from jax.experimental import pallas as pl
from jax.experimental.pallas import tpu as pltpu
from jax.experimental.pallas import tpu_sc as plsc

L = 16
BE = 128


@functools.lru_cache(maxsize=None)
def _make_sc_kernel(N, E, D):
    assert E % BE == 0
    NB = E // BE
    info = plsc.get_sparse_core_info()
    NW = info.num_cores * info.num_subcores
    n_iter = (NB + NW - 1) // NW
    NC = info.num_cores
    mesh = plsc.VectorSubcoreMesh(core_axis_name="c", subcore_axis_name="s")

    @functools.partial(
        pl.kernel,
        out_type=jax.ShapeDtypeStruct((E, 3 * D), jnp.float32),
        mesh=mesh,
        scratch_types=[
            pltpu.VMEM((BE,), jnp.int32),
            pltpu.VMEM((BE,), jnp.int32),
            pltpu.VMEM((BE, D), jnp.float32),
            pltpu.VMEM((5, BE), jnp.float32),
            pltpu.VMEM((3 * BE,), jnp.float32),
            pltpu.VMEM((BE, 3 * D), jnp.float32),
            pltpu.SemaphoreType.DMA,
        ],
    )
    def body(x_hbm, geo_hbm, idx_hbm, out_hbm,
             idx_raw, idx_safe, rows, geo, abc, outb, sem):
        wid = lax.axis_index("s") * NC + lax.axis_index("c")

        @pl.loop(0, n_iter)
        def _batches(i):
            bid = wid + i * NW

            @pl.when(bid < NB)
            def _():
                base = bid * BE
                pltpu.sync_copy(idx_hbm.at[pl.ds(base, BE)], idx_raw)
                for j in range(BE // L):
                    sl = pl.ds(j * L, L)
                    idx_safe[sl] = jnp.maximum(idx_raw[sl], 0)
                gather = pltpu.async_copy(x_hbm.at[idx_safe], rows, sem)
                pltpu.sync_copy(geo_hbm.at[:, pl.ds(base, BE)], geo)
                for j in range(BE // L):
                    sl = pl.ds(j * L, L)
                    mask = jnp.where(idx_raw[sl] < 0, 0.0, 1.0)
                    scale = mask / (geo[0, sl] + 1e-6)
                    st = geo[3, sl]
                    abc[pl.ds(j * L, L)] = st * geo[2, sl] * scale
                    abc[pl.ds(BE + j * L, L)] = st * geo[1, sl] * scale
                    abc[pl.ds(2 * BE + j * L, L)] = geo[4, sl] * scale
                gather.wait()

                @pl.loop(0, BE // L)
                def _groups(g):
                    av = abc[pl.ds(g * L, L)]
                    bv = abc[pl.ds(BE + g * L, L)]
                    cv = abc[pl.ds(2 * BE + g * L, L)]
                    for l in range(L):
                        e = g * L + l
                        a = jnp.full((L,), av[l], jnp.float32)
                        b = jnp.full((L,), bv[l], jnp.float32)
                        c = jnp.full((L,), cv[l], jnp.float32)
                        for cc in range(D // L):
                            r = rows[e, pl.ds(cc * L, L)]
                            outb[e, pl.ds(cc * L, L)] = r * a
                            outb[e, pl.ds(D + cc * L, L)] = r * b
                            outb[e, pl.ds(2 * D + cc * L, L)] = r * c

                pltpu.sync_copy(outb, out_hbm.at[pl.ds(base, BE), :])

    return body


def kernel(x, geo_features, neighbor_indices):
    N, D = x.shape
    _, K, _ = geo_features.shape
    E = N * K
    idx = neighbor_indices.reshape(E).astype(jnp.int32)
    geo_t = geo_features.reshape(E, 5).T
    out = _make_sc_kernel(N, E, D)(x, geo_t, idx)
    return out.reshape(N, K, 3 * D)


# speedup vs baseline: 2.1391x; 1.4239x over previous
import functools

import jax
import jax.numpy as jnp
from jax import lax
from jax.experimental import pallas as pl
from jax.experimental.pallas import tpu as pltpu
from jax.experimental.pallas import tpu_sc as plsc

L = 16
BE = 64


@functools.lru_cache(maxsize=None)
def _make_sc_kernel(N, E, D):
    assert E % BE == 0
    NB = E // BE
    info = plsc.get_sparse_core_info()
    NC = info.num_cores
    NW = NC * info.num_subcores
    T = (NB + NW - 1) // NW
    T2 = T + (T % 2)
    mesh = plsc.VectorSubcoreMesh(core_axis_name="c", subcore_axis_name="s")

    @functools.partial(
        pl.kernel,
        out_type=jax.ShapeDtypeStruct((E, 3 * D), jnp.float32),
        mesh=mesh,
        scratch_types=[
            pltpu.VMEM((T2 * BE,), jnp.int32),
            pltpu.VMEM((T2 * 5 * BE,), jnp.float32),
            pltpu.VMEM((2, BE, D), jnp.float32),
            pltpu.VMEM((2, BE, 3 * D), jnp.float32),
            pltpu.SemaphoreType.DMA,
            pltpu.SemaphoreType.DMA,
            pltpu.SemaphoreType.DMA,
            pltpu.SemaphoreType.DMA,
            pltpu.SemaphoreType.DMA,
            pltpu.SemaphoreType.DMA,
        ],
    )
    def body(x_hbm, geo_hbm, idx_hbm, out_hbm,
             idx_all, geo_all, rows, outb, isem, msem, gsem0, gsem1,
             osem0, osem1):
        wid = lax.axis_index("s") * NC + lax.axis_index("c")
        gsem = (gsem0, gsem1)
        osem = (osem0, osem1)

        def valid(t):
            return wid + t * NW < NB

        def base_of(t):
            return (wid + t * NW) * BE

        @pl.loop(0, T2)
        def _fire(t):
            @pl.when(valid(t))
            def _():
                base = base_of(t)
                pltpu.async_copy(idx_hbm.at[pl.ds(base, BE)],
                                 idx_all.at[pl.ds(t * BE, BE)], isem)
                for f in range(5):
                    pltpu.async_copy(
                        geo_hbm.at[pl.ds(f * E + base, BE)],
                        geo_all.at[pl.ds((t * 5 + f) * BE, BE)], msem)

        @pl.loop(0, T2)
        def _scalars(t):
            @pl.when(valid(t))
            def _():
                base = base_of(t)
                pltpu.make_async_copy(idx_hbm.at[pl.ds(base, BE)],
                                      idx_all.at[pl.ds(t * BE, BE)],
                                      isem).wait()
                for f in range(5):
                    pltpu.make_async_copy(
                        geo_hbm.at[pl.ds(f * E + base, BE)],
                        geo_all.at[pl.ds((t * 5 + f) * BE, BE)], msem).wait()
                for j in range(BE // L):
                    ix = pl.ds(t * BE + j * L, L)
                    gf = [pl.ds((t * 5 + f) * BE + j * L, L)
                          for f in range(5)]
                    iv = idx_all[ix]
                    idx_all[ix] = jnp.maximum(iv, 0)
                    mask = jnp.where(iv < 0, 0.0, 1.0)
                    scale = mask / (geo_all[gf[0]] + 1e-6)
                    st = geo_all[gf[3]]
                    a = st * geo_all[gf[2]] * scale
                    b = st * geo_all[gf[1]] * scale
                    c = geo_all[gf[4]] * scale
                    geo_all[gf[0]] = a
                    geo_all[gf[1]] = b
                    geo_all[gf[2]] = c

        def fire_gather(t, buf):
            @pl.when(valid(t))
            def _():
                pltpu.async_copy(x_hbm.at[idx_all.at[pl.ds(t * BE, BE)]],
                                 rows.at[buf], gsem[buf])

        fire_gather(0, 0)

        @pl.loop(0, T2, step=2)
        def _steps(i):
            for b in (0, 1):
                t = i + b

                @pl.when(t + 1 < T2)
                def _():
                    fire_gather(t + 1, 1 - b)

                @pl.when(valid(t))
                def _():
                    pltpu.make_async_copy(
                        x_hbm.at[idx_all.at[pl.ds(t * BE, BE)]],
                        rows.at[b], gsem[b]).wait()

                    @pl.when(t >= 2)
                    def _():
                        pltpu.make_async_copy(
                            outb.at[b], out_hbm.at[pl.ds(0, BE), :],
                            osem[b]).wait()

                    @pl.loop(0, BE // L)
                    def _groups(g):
                        av = geo_all[pl.ds(t * 5 * BE + g * L, L)]
                        bv = geo_all[pl.ds((t * 5 + 1) * BE + g * L, L)]
                        cv = geo_all[pl.ds((t * 5 + 2) * BE + g * L, L)]
                        for l in range(L):
                            e = g * L + l
                            a = jnp.full((L,), av[l], jnp.float32)
                            bb = jnp.full((L,), bv[l], jnp.float32)
                            c = jnp.full((L,), cv[l], jnp.float32)
                            for cc in range(D // L):
                                r = rows[b, e, pl.ds(cc * L, L)]
                                outb[b, e, pl.ds(cc * L, L)] = r * a
                                outb[b, e, pl.ds(D + cc * L, L)] = r * bb
                                outb[b, e, pl.ds(2 * D + cc * L, L)] = r * c

                    pltpu.async_copy(outb.at[b],
                                     out_hbm.at[pl.ds(base_of(t), BE), :],
                                     osem[b])

        assert NB >= 2 * NW
        for b in (0, 1):
            pltpu.make_async_copy(outb.at[b], out_hbm.at[pl.ds(0, BE), :],
                                  osem[b]).wait()

    return body


def kernel(x, geo_features, neighbor_indices):
    N, D = x.shape
    _, K, _ = geo_features.shape
    E = N * K
    idx = neighbor_indices.reshape(E).astype(jnp.int32)
    geo_t = geo_features.reshape(E, 5).T.reshape(5 * E)
    out = _make_sc_kernel(N, E, D)(x, geo_t, idx)
    return out.reshape(N, K, 3 * D)


# speedup vs baseline: 2.1615x; 1.0104x over previous
import functools

import jax
import jax.numpy as jnp
from jax import lax
from jax.experimental import pallas as pl
from jax.experimental.pallas import tpu as pltpu
from jax.experimental.pallas import tpu_sc as plsc

L = 16
BE = 80


@functools.lru_cache(maxsize=None)
def _make_sc_kernel(N, E, D):
    assert E % BE == 0
    NB = E // BE
    info = plsc.get_sparse_core_info()
    NC = info.num_cores
    NW = NC * info.num_subcores
    T = (NB + NW - 1) // NW
    T2 = T + (T % 2)
    mesh = plsc.VectorSubcoreMesh(core_axis_name="c", subcore_axis_name="s")

    @functools.partial(
        pl.kernel,
        out_type=jax.ShapeDtypeStruct((E, 3 * D), jnp.float32),
        mesh=mesh,
        scratch_types=[
            pltpu.VMEM((T2 * BE,), jnp.int32),
            pltpu.VMEM((T2 * 5 * BE,), jnp.float32),
            pltpu.VMEM((2, BE, D), jnp.float32),
            pltpu.VMEM((2, BE, 3 * D), jnp.float32),
            pltpu.SemaphoreType.DMA,
            pltpu.SemaphoreType.DMA,
            pltpu.SemaphoreType.DMA,
            pltpu.SemaphoreType.DMA,
            pltpu.SemaphoreType.DMA,
            pltpu.SemaphoreType.DMA,
        ],
    )
    def body(x_hbm, geo_hbm, idx_hbm, out_hbm,
             idx_all, geo_all, rows, outb, isem, msem, gsem0, gsem1,
             osem0, osem1):
        wid = lax.axis_index("s") * NC + lax.axis_index("c")
        gsem = (gsem0, gsem1)
        osem = (osem0, osem1)

        def valid(t):
            return wid + t * NW < NB

        def base_of(t):
            return (wid + t * NW) * BE

        @pl.loop(0, T2)
        def _fire(t):
            @pl.when(valid(t))
            def _():
                base = base_of(t)
                pltpu.async_copy(idx_hbm.at[pl.ds(base, BE)],
                                 idx_all.at[pl.ds(t * BE, BE)], isem)
                for f in range(5):
                    pltpu.async_copy(
                        geo_hbm.at[pl.ds(f * E + base, BE)],
                        geo_all.at[pl.ds((t * 5 + f) * BE, BE)], msem)

        @pl.loop(0, T2)
        def _scalars(t):
            @pl.when(valid(t))
            def _():
                base = base_of(t)
                pltpu.make_async_copy(idx_hbm.at[pl.ds(base, BE)],
                                      idx_all.at[pl.ds(t * BE, BE)],
                                      isem).wait()
                for f in range(5):
                    pltpu.make_async_copy(
                        geo_hbm.at[pl.ds(f * E + base, BE)],
                        geo_all.at[pl.ds((t * 5 + f) * BE, BE)], msem).wait()
                for j in range(BE // L):
                    ix = pl.ds(t * BE + j * L, L)
                    gf = [pl.ds((t * 5 + f) * BE + j * L, L)
                          for f in range(5)]
                    iv = idx_all[ix]
                    idx_all[ix] = jnp.maximum(iv, 0)
                    mask = jnp.where(iv < 0, 0.0, 1.0)
                    scale = mask / (geo_all[gf[0]] + 1e-6)
                    st = geo_all[gf[3]]
                    a = st * geo_all[gf[2]] * scale
                    b = st * geo_all[gf[1]] * scale
                    c = geo_all[gf[4]] * scale
                    geo_all[gf[0]] = a
                    geo_all[gf[1]] = b
                    geo_all[gf[2]] = c

        def fire_gather(t, buf):
            @pl.when(valid(t))
            def _():
                pltpu.async_copy(x_hbm.at[idx_all.at[pl.ds(t * BE, BE)]],
                                 rows.at[buf], gsem[buf])

        fire_gather(0, 0)

        @pl.loop(0, T2, step=2)
        def _steps(i):
            for b in (0, 1):
                t = i + b

                @pl.when(t + 1 < T2)
                def _():
                    fire_gather(t + 1, 1 - b)

                @pl.when(valid(t))
                def _():
                    pltpu.make_async_copy(
                        x_hbm.at[idx_all.at[pl.ds(t * BE, BE)]],
                        rows.at[b], gsem[b]).wait()

                    @pl.when(t >= 2)
                    def _():
                        pltpu.make_async_copy(
                            outb.at[b], out_hbm.at[pl.ds(0, BE), :],
                            osem[b]).wait()

                    @pl.loop(0, BE // L)
                    def _groups(g):
                        av = geo_all[pl.ds(t * 5 * BE + g * L, L)]
                        bv = geo_all[pl.ds((t * 5 + 1) * BE + g * L, L)]
                        cv = geo_all[pl.ds((t * 5 + 2) * BE + g * L, L)]
                        for l in range(L):
                            e = g * L + l
                            a = jnp.full((L,), av[l], jnp.float32)
                            bb = jnp.full((L,), bv[l], jnp.float32)
                            c = jnp.full((L,), cv[l], jnp.float32)
                            for cc in range(D // L):
                                r = rows[b, e, pl.ds(cc * L, L)]
                                outb[b, e, pl.ds(cc * L, L)] = r * a
                                outb[b, e, pl.ds(D + cc * L, L)] = r * bb
                                outb[b, e, pl.ds(2 * D + cc * L, L)] = r * c

                    pltpu.async_copy(outb.at[b],
                                     out_hbm.at[pl.ds(base_of(t), BE), :],
                                     osem[b])

        assert NB >= 2 * NW
        for b in (0, 1):
            pltpu.make_async_copy(outb.at[b], out_hbm.at[pl.ds(0, BE), :],
                                  osem[b]).wait()

    return body


def kernel(x, geo_features, neighbor_indices):
    N, D = x.shape
    _, K, _ = geo_features.shape
    E = N * K
    idx = neighbor_indices.reshape(E).astype(jnp.int32)
    geo_t = geo_features.reshape(E, 5).T.reshape(5 * E)
    out = _make_sc_kernel(N, E, D)(x, geo_t, idx)
    return out.reshape(N, K, 3 * D)
